# baseline (speedup 1.0000x reference)
import functools

import jax
import jax.numpy as jnp
from jax import lax
from jax.experimental import pallas as pl
from jax.experimental.pallas import tpu as pltpu
from jax.experimental.pallas import tpu_sc as plsc

N = 10000
D = 128
E = 320000
DH = D // 2
NC = 2
NS = 16
CH = 128
NCHUNK = E // CH
K = -(-NCHUNK // NS)
KL = NCHUNK - (NS - 1) * K
KH = (K + 1) // 2
ROWS_PER_TILE = 625
RFULL = (ROWS_PER_TILE // CH) * CH
RTAIL = ROWS_PER_TILE - RFULL


def _fill(ref, rows, cols, value):
    v = jnp.full((16,), value, jnp.float32)

    def row(i, _):
        def col(j, _):
            ref[i, pl.ds(j * 16, 16)] = v
            return 0
        return lax.fori_loop(0, cols // 16, col, 0)

    lax.fori_loop(0, rows, row, 0)


def _make_sc_agg(with_deg: bool):
    mesh = plsc.VectorSubcoreMesh(core_axis_name="c", subcore_axis_name="s",
                                  num_cores=NC, num_subcores=NS)
    out_type = [jax.ShapeDtypeStruct((N, D), jnp.float32)]
    scratch = [
        pltpu.VMEM((K, CH), jnp.int32),
        pltpu.VMEM((K, CH), jnp.int32),
        pltpu.VMEM(((3 if with_deg else 4) * CH, DH), jnp.float32),
        pltpu.VMEM((CH, DH), jnp.float32),
        pltpu.VMEM_SHARED((N, DH), jnp.float32),
        pltpu.SemaphoreType.DMA,
        pltpu.SemaphoreType.DMA,
    ]
    if with_deg:
        out_type.append(jax.ShapeDtypeStruct((NC, N, 16), jnp.float32))
        scratch += [
            pltpu.VMEM((CH, 16), jnp.float32),
            pltpu.VMEM((CH, 16), jnp.float32),
            pltpu.VMEM_SHARED((N, 16), jnp.float32),
        ]

    def body(z2_hbm, src_hbm, dst_hbm, out_hbm, *rest):
        if with_deg:
            (deg_hbm, idx_s, idx_d, rows, zrows, acc_sh, sem, sem_s,
             ones16, zdeg, deg_sh) = rest
        else:
            (idx_s, idx_d, rows, zrows, acc_sh, sem, sem_s) = rest
        cid = lax.axis_index("c")
        sid = lax.axis_index("s")
        base = sid * ROWS_PER_TILE
        kt = jnp.where(sid == NS - 1, KL, K)

        _fill(zrows, CH, DH, 0.0)
        if with_deg:
            _fill(ones16, CH, 16, 1.0)
            _fill(zdeg, CH, 16, 0.0)

        def zacc(t, _):
            pltpu.sync_copy(zrows, acc_sh.at[pl.ds(base + t * CH, CH)])
            if with_deg:
                pltpu.sync_copy(zdeg, deg_sh.at[pl.ds(base + t * CH, CH)])
            return 0
        lax.fori_loop(0, RFULL // CH, zacc, 0)
        pltpu.sync_copy(zrows.at[pl.ds(0, RTAIL)],
                        acc_sh.at[pl.ds(base + RFULL, RTAIL)])
        if with_deg:
            pltpu.sync_copy(zdeg.at[pl.ds(0, RTAIL)],
                            deg_sh.at[pl.ds(base + RFULL, RTAIL)])

        @pl.when(sid < NS - 1)
        def _():
            pltpu.sync_copy(src_hbm.at[pl.ds(sid * K, K)], idx_s)
            pltpu.sync_copy(dst_hbm.at[pl.ds(sid * K, K)], idx_d)

        @pl.when(sid == NS - 1)
        def _():
            pltpu.sync_copy(src_hbm.at[pl.ds((NS - 1) * K, KL)],
                            idx_s.at[pl.ds(0, KL)])
            pltpu.sync_copy(dst_hbm.at[pl.ds((NS - 1) * K, KL)],
                            idx_d.at[pl.ds(0, KL)])

        def txrow(i, _):
            def txcol(c2, _):
                sl = idx_s[i, pl.ds(c2 * 16, 16)]
                idx_s[i, pl.ds(c2 * 16, 16)] = sl * 2 + cid
                return 0
            return lax.fori_loop(0, CH // 16, txcol, 0)
        lax.fori_loop(0, kt, txrow, 0)
        plsc.subcore_barrier()

        def make_loop(deg_lo, deg_hi):
            RB = 3 if with_deg else 4
            G = 2
            SO = RB - G

            def run():
                for b in range(G):
                    pltpu.async_copy(z2_hbm.at[idx_s.at[b]],
                                     rows.at[pl.ds(b * CH, CH)], sem)

                def step(j, _):
                    slot = lax.rem(j, RB)
                    buf = rows.at[pl.ds(slot * CH, CH)]
                    pltpu.make_async_copy(z2_hbm.at[idx_s.at[j]], buf,
                                          sem).wait()
                    @pl.when(j >= SO)
                    def _():
                        pslot = lax.rem(j + G, RB)
                        pbuf = rows.at[pl.ds(pslot * CH, CH)]
                        pltpu.make_async_copy(
                            pbuf, acc_sh.at[idx_d.at[j - SO]], sem_s).wait()
                    @pl.when(j + G < kt)
                    def _():
                        nslot = lax.rem(j + G, RB)
                        pltpu.async_copy(
                            z2_hbm.at[idx_s.at[j + G]],
                            rows.at[pl.ds(nslot * CH, CH)], sem)
                    pltpu.async_copy(buf, acc_sh.at[idx_d.at[j]], sem_s,
                                     add=True)
                    if with_deg:
                        @pl.when(jnp.logical_and(j >= deg_lo, j < deg_hi))
                        def _():
                            pltpu.sync_copy(ones16, deg_sh.at[idx_d.at[j]],
                                            add=True)
                    return 0
                lax.fori_loop(0, kt, step, 0)
                for t in range(SO):
                    pltpu.make_async_copy(
                        rows.at[pl.ds(t * CH, CH)],
                        acc_sh.at[idx_d.at[t]], sem_s).wait()
            return run

        @pl.when(cid == 0)
        def _():
            make_loop(0, KH)()

        @pl.when(cid == 1)
        def _():
            make_loop(KH, K)()

        plsc.subcore_barrier()

        pltpu.sync_copy(
            acc_sh.at[pl.ds(base, ROWS_PER_TILE)],
            out_hbm.at[pl.ds(base, ROWS_PER_TILE), pl.ds(cid * DH, DH)])
        if with_deg:
            pltpu.sync_copy(deg_sh.at[pl.ds(base, ROWS_PER_TILE)],
                            deg_hbm.at[cid].at[pl.ds(base, ROWS_PER_TILE)])

    return pl.kernel(
        body, out_type=tuple(out_type), mesh=mesh, scratch_types=scratch,
        compiler_params=pltpu.CompilerParams(use_tc_tiling_on_sc=False))


_make_sc_agg = functools.cache(_make_sc_agg)


def _dense_body(z_ref, p_ref, g_ref, w_ref, b_ref, o_ref):
    deg = jnp.maximum(g_ref[0, :, 0:1] + g_ref[1, :, 0:1], 1.0)
    zn = p_ref[...] / deg
    x = (jnp.dot(z_ref[...], w_ref[:D], preferred_element_type=jnp.float32)
         + jnp.dot(zn, w_ref[D:], preferred_element_type=jnp.float32)
         + b_ref[...])
    h = jax.nn.sigmoid(x)
    nrm = jnp.sqrt(jnp.sum(h * h, axis=1, keepdims=True))
    o_ref[...] = h / jnp.maximum(nrm, 1e-12)


_R = 1000
_tc_dense = pl.pallas_call(
    _dense_body,
    grid=(N // _R,),
    in_specs=[
        pl.BlockSpec((_R, D), lambda i: (i, 0)),
        pl.BlockSpec((_R, D), lambda i: (i, 0)),
        pl.BlockSpec((NC, _R, 16), lambda i: (0, i, 0)),
        pl.BlockSpec((2 * D, D), lambda i: (0, 0)),
        pl.BlockSpec((1, D), lambda i: (0, 0)),
    ],
    out_specs=pl.BlockSpec((_R, D), lambda i: (i, 0)),
    out_shape=jax.ShapeDtypeStruct((N, D), jnp.float32),
)


def kernel(Z, edge_index, W0, b0, W1, b1):
    src2 = edge_index[0].reshape(NCHUNK, CH)
    dst2 = edge_index[1].reshape(NCHUNK, CH)

    agg1, degp = _make_sc_agg(True)(Z.reshape(2 * N, DH), src2, dst2)
    Z1 = _tc_dense(Z, agg1, degp, W0, b0.reshape(1, D))
    (agg2,) = _make_sc_agg(False)(Z1.reshape(2 * N, DH), src2, dst2)
    return _tc_dense(Z1, agg2, degp, W1, b1.reshape(1, D))


# speedup vs baseline: 1.0704x; 1.0704x over previous
import functools

import jax
import jax.numpy as jnp
from jax import lax
from jax.experimental import pallas as pl
from jax.experimental.pallas import tpu as pltpu
from jax.experimental.pallas import tpu_sc as plsc

N = 10000
D = 128
E = 320000
DH = D // 2
NC = 2
NS = 16
CH = 128
NCHUNK = E // CH
K = -(-NCHUNK // NS)
KL = NCHUNK - (NS - 1) * K
KH = (K + 1) // 2
ROWS_PER_TILE = 625
RFULL = (ROWS_PER_TILE // CH) * CH
RTAIL = ROWS_PER_TILE - RFULL


def _fill(ref, rows, cols, value):
    v = jnp.full((16,), value, jnp.float32)

    def row(i, _):
        def col(j, _):
            ref[i, pl.ds(j * 16, 16)] = v
            return 0
        return lax.fori_loop(0, cols // 16, col, 0)

    lax.fori_loop(0, rows, row, 0)


def _make_sc_agg(with_deg: bool):
    mesh = plsc.VectorSubcoreMesh(core_axis_name="c", subcore_axis_name="s",
                                  num_cores=NC, num_subcores=NS)
    out_type = [jax.ShapeDtypeStruct((N, D), jnp.float32)]
    scratch = [
        pltpu.VMEM((K, CH), jnp.int32),
        pltpu.VMEM((K, CH), jnp.int32),
        pltpu.VMEM(((3 if with_deg else 4) * CH, DH), jnp.float32),
        pltpu.VMEM((CH, DH), jnp.float32),
        pltpu.VMEM_SHARED((N, DH), jnp.float32),
        pltpu.SemaphoreType.DMA,
        pltpu.SemaphoreType.DMA,
    ]
    if with_deg:
        out_type.append(jax.ShapeDtypeStruct((NC, N, 16), jnp.float32))
        scratch += [
            pltpu.VMEM((CH, 16), jnp.float32),
            pltpu.VMEM((CH, 16), jnp.float32),
            pltpu.VMEM_SHARED((N, 16), jnp.float32),
        ]

    def body(z2_hbm, src_hbm, dst_hbm, out_hbm, *rest):
        if with_deg:
            (deg_hbm, idx_s, idx_d, rows, zrows, acc_sh, sem, sem_s,
             ones16, zdeg, deg_sh) = rest
        else:
            (idx_s, idx_d, rows, zrows, acc_sh, sem, sem_s) = rest
        cid = lax.axis_index("c")
        sid = lax.axis_index("s")
        base = sid * ROWS_PER_TILE
        kt = jnp.where(sid == NS - 1, KL, K)

        _fill(zrows, CH, DH, 0.0)
        if with_deg:
            _fill(ones16, CH, 16, 1.0)
            _fill(zdeg, CH, 16, 0.0)

        def zacc(t, _):
            pltpu.sync_copy(zrows, acc_sh.at[pl.ds(base + t * CH, CH)])
            if with_deg:
                pltpu.sync_copy(zdeg, deg_sh.at[pl.ds(base + t * CH, CH)])
            return 0
        lax.fori_loop(0, RFULL // CH, zacc, 0)
        pltpu.sync_copy(zrows.at[pl.ds(0, RTAIL)],
                        acc_sh.at[pl.ds(base + RFULL, RTAIL)])
        if with_deg:
            pltpu.sync_copy(zdeg.at[pl.ds(0, RTAIL)],
                            deg_sh.at[pl.ds(base + RFULL, RTAIL)])

        @pl.when(sid < NS - 1)
        def _():
            pltpu.sync_copy(src_hbm.at[pl.ds(sid * K, K)], idx_s)
            pltpu.sync_copy(dst_hbm.at[pl.ds(sid * K, K)], idx_d)

        @pl.when(sid == NS - 1)
        def _():
            pltpu.sync_copy(src_hbm.at[pl.ds((NS - 1) * K, KL)],
                            idx_s.at[pl.ds(0, KL)])
            pltpu.sync_copy(dst_hbm.at[pl.ds((NS - 1) * K, KL)],
                            idx_d.at[pl.ds(0, KL)])

        def txrow(i, _):
            def txcol(c2, _):
                sl = idx_s[i, pl.ds(c2 * 16, 16)]
                idx_s[i, pl.ds(c2 * 16, 16)] = sl * 2 + cid
                return 0
            return lax.fori_loop(0, CH // 16, txcol, 0)
        lax.fori_loop(0, kt, txrow, 0)
        plsc.subcore_barrier()

        def make_loop(deg_lo, deg_hi):
            RB = 3 if with_deg else 4
            G = 2 if with_deg else 3
            SO = RB - G

            def run():
                for b in range(G):
                    pltpu.async_copy(z2_hbm.at[idx_s.at[b]],
                                     rows.at[pl.ds(b * CH, CH)], sem)

                def step(j, _):
                    slot = lax.rem(j, RB)
                    buf = rows.at[pl.ds(slot * CH, CH)]
                    pltpu.make_async_copy(z2_hbm.at[idx_s.at[j]], buf,
                                          sem).wait()
                    @pl.when(j >= SO)
                    def _():
                        pslot = lax.rem(j + G, RB)
                        pbuf = rows.at[pl.ds(pslot * CH, CH)]
                        pltpu.make_async_copy(
                            pbuf, acc_sh.at[idx_d.at[j - SO]], sem_s).wait()
                    @pl.when(j + G < kt)
                    def _():
                        nslot = lax.rem(j + G, RB)
                        pltpu.async_copy(
                            z2_hbm.at[idx_s.at[j + G]],
                            rows.at[pl.ds(nslot * CH, CH)], sem)
                    pltpu.async_copy(buf, acc_sh.at[idx_d.at[j]], sem_s,
                                     add=True)
                    if with_deg:
                        @pl.when(jnp.logical_and(j >= deg_lo, j < deg_hi))
                        def _():
                            pltpu.sync_copy(ones16, deg_sh.at[idx_d.at[j]],
                                            add=True)
                    return 0
                lax.fori_loop(0, kt, step, 0)
                for t in range(SO):
                    pltpu.make_async_copy(
                        rows.at[pl.ds(t * CH, CH)],
                        acc_sh.at[idx_d.at[t]], sem_s).wait()
            return run

        @pl.when(cid == 0)
        def _():
            make_loop(0, KH)()

        @pl.when(cid == 1)
        def _():
            make_loop(KH, K)()

        plsc.subcore_barrier()

        pltpu.sync_copy(
            acc_sh.at[pl.ds(base, ROWS_PER_TILE)],
            out_hbm.at[pl.ds(base, ROWS_PER_TILE), pl.ds(cid * DH, DH)])
        if with_deg:
            pltpu.sync_copy(deg_sh.at[pl.ds(base, ROWS_PER_TILE)],
                            deg_hbm.at[cid].at[pl.ds(base, ROWS_PER_TILE)])

    return pl.kernel(
        body, out_type=tuple(out_type), mesh=mesh, scratch_types=scratch,
        compiler_params=pltpu.CompilerParams(use_tc_tiling_on_sc=False))


_make_sc_agg = functools.cache(_make_sc_agg)


def _dense_body(z_ref, p_ref, g_ref, w_ref, b_ref, o_ref):
    deg = jnp.maximum(g_ref[0, :, 0:1] + g_ref[1, :, 0:1], 1.0)
    zn = p_ref[...] / deg
    x = (jnp.dot(z_ref[...], w_ref[:D], preferred_element_type=jnp.float32)
         + jnp.dot(zn, w_ref[D:], preferred_element_type=jnp.float32)
         + b_ref[...])
    h = jax.nn.sigmoid(x)
    nrm = jnp.sqrt(jnp.sum(h * h, axis=1, keepdims=True))
    o_ref[...] = h / jnp.maximum(nrm, 1e-12)


_R = 2000
_tc_dense = pl.pallas_call(
    _dense_body,
    grid=(N // _R,),
    in_specs=[
        pl.BlockSpec((_R, D), lambda i: (i, 0)),
        pl.BlockSpec((_R, D), lambda i: (i, 0)),
        pl.BlockSpec((NC, _R, 16), lambda i: (0, i, 0)),
        pl.BlockSpec((2 * D, D), lambda i: (0, 0)),
        pl.BlockSpec((1, D), lambda i: (0, 0)),
    ],
    out_specs=pl.BlockSpec((_R, D), lambda i: (i, 0)),
    out_shape=jax.ShapeDtypeStruct((N, D), jnp.float32),
)


def kernel(Z, edge_index, W0, b0, W1, b1):
    src2 = edge_index[0].reshape(NCHUNK, CH)
    dst2 = edge_index[1].reshape(NCHUNK, CH)

    agg1, degp = _make_sc_agg(True)(Z.reshape(2 * N, DH), src2, dst2)
    Z1 = _tc_dense(Z, agg1, degp, W0, b0.reshape(1, D))
    (agg2,) = _make_sc_agg(False)(Z1.reshape(2 * N, DH), src2, dst2)
    return _tc_dense(Z1, agg2, degp, W1, b1.reshape(1, D))
